# baseline (speedup 1.0000x reference)
import functools

import jax
import jax.numpy as jnp
import numpy as np
from jax.experimental import pallas as pl
from jax.experimental.pallas import tpu as pltpu

_INV12 = np.float32(1.0 / 12.0)
_QB = 512


def _fused_kernel(x_ref, hp_ref, out_ref, acc_ref, *, K, D):
    h = pl.program_id(1)
    q = pl.program_id(2)
    nh = pl.num_programs(1)
    nq = pl.num_programs(2)

    @pl.when((h == 0) & (q == 0))
    def _():
        acc_ref[...] = jnp.zeros_like(acc_ref)

    y = x_ref[0, 0] * _INV12
    acc = acc_ref[...]
    for t in range(_QB // 8):
        acc = acc + y[8 * t : 8 * t + 8, :]
    acc_ref[...] = acc

    @pl.when((h == nh - 1) & (q == nq - 1))
    def _():
        a = acc_ref[...]
        t1 = a[0:4, :] + a[4:8, :]
        t2 = t1[0:2, :] + t1[2:4, :]
        imp = t2[0:1, :] + t2[1:2, :]
        r2 = 2 * jax.lax.broadcasted_iota(jnp.int32, (K, 2 * K), 0)
        c = jax.lax.broadcasted_iota(jnp.int32, (K, 2 * K), 1)
        msign = jnp.where(c == r2 + 1, np.float32(1.0), np.float32(0.0)) - jnp.where(
            c == r2, np.float32(1.0), np.float32(0.0)
        )
        d = jax.lax.dot_general(
            msign,
            imp,
            (((1,), (1,)), ((), ())),
            preferred_element_type=jnp.float32,
        )
        row = jax.lax.broadcasted_iota(jnp.int32, (K, 1), 0)
        bit = (d > 0) & (row > 0)
        hp = hp_ref[0]
        out_ref[0] = jnp.where(bit, hp[:, D:], hp[:, :D])


def kernel(hidden_states, self_attention_scores, key_layer, tome_size):
    B, L, D = hidden_states.shape
    H = self_attention_scores.shape[1]
    K = L // 2

    hidden_pairs = hidden_states.reshape(B, K, 2 * D)

    final_token = pl.pallas_call(
        functools.partial(_fused_kernel, K=K, D=D),
        grid=(B, H, L // _QB),
        in_specs=[
            pl.BlockSpec((1, 1, _QB, L), lambda b, h, q: (b, h, q, 0)),
            pl.BlockSpec((1, K, 2 * D), lambda b, h, q: (b, 0, 0)),
        ],
        out_specs=pl.BlockSpec((1, K, D), lambda b, h, q: (b, 0, 0)),
        out_shape=jax.ShapeDtypeStruct((B, K, D), jnp.float32),
        scratch_shapes=[pltpu.VMEM((8, L), jnp.float32)],
    )(self_attention_scores, hidden_pairs)

    tome_size_out = jnp.ones((B, K, 1), dtype=jnp.float32)
    return (final_token, tome_size_out)


# speedup vs baseline: 1.0798x; 1.0798x over previous
import functools

import jax
import jax.numpy as jnp
import numpy as np
from jax.experimental import pallas as pl
from jax.experimental.pallas import tpu as pltpu

_INV12 = np.float32(1.0 / 12.0)
_QB = 1024


def _fused_kernel(x_ref, hp_ref, out_ref, acc_ref, *, K, D):
    h = pl.program_id(1)
    q = pl.program_id(2)
    nh = pl.num_programs(1)
    nq = pl.num_programs(2)

    @pl.when((h == 0) & (q == 0))
    def _():
        acc_ref[...] = jnp.zeros_like(acc_ref)

    y = x_ref[0, 0] * _INV12
    acc = acc_ref[...]
    for t in range(_QB // 8):
        acc = acc + y[8 * t : 8 * t + 8, :]
    acc_ref[...] = acc

    @pl.when((h == nh - 1) & (q == nq - 1))
    def _():
        a = acc_ref[...]
        t1 = a[0:4, :] + a[4:8, :]
        t2 = t1[0:2, :] + t1[2:4, :]
        imp = t2[0:1, :] + t2[1:2, :]
        r2 = 2 * jax.lax.broadcasted_iota(jnp.int32, (K, 2 * K), 0)
        c = jax.lax.broadcasted_iota(jnp.int32, (K, 2 * K), 1)
        msign = jnp.where(c == r2 + 1, np.float32(1.0), np.float32(0.0)) - jnp.where(
            c == r2, np.float32(1.0), np.float32(0.0)
        )
        d = jax.lax.dot_general(
            msign,
            imp,
            (((1,), (1,)), ((), ())),
            preferred_element_type=jnp.float32,
        )
        row = jax.lax.broadcasted_iota(jnp.int32, (K, 1), 0)
        bit = (d > 0) & (row > 0)
        hp = hp_ref[0]
        out_ref[0] = jnp.where(bit, hp[:, D:], hp[:, :D])


def kernel(hidden_states, self_attention_scores, key_layer, tome_size):
    B, L, D = hidden_states.shape
    H = self_attention_scores.shape[1]
    K = L // 2

    hidden_pairs = hidden_states.reshape(B, K, 2 * D)

    final_token = pl.pallas_call(
        functools.partial(_fused_kernel, K=K, D=D),
        grid=(B, H, L // _QB),
        in_specs=[
            pl.BlockSpec((1, 1, _QB, L), lambda b, h, q: (b, h, q, 0)),
            pl.BlockSpec((1, K, 2 * D), lambda b, h, q: (b, 0, 0)),
        ],
        out_specs=pl.BlockSpec((1, K, D), lambda b, h, q: (b, 0, 0)),
        out_shape=jax.ShapeDtypeStruct((B, K, D), jnp.float32),
        scratch_shapes=[pltpu.VMEM((8, L), jnp.float32)],
    )(self_attention_scores, hidden_pairs)

    tome_size_out = jnp.ones((B, K, 1), dtype=jnp.float32)
    return (final_token, tome_size_out)
